# Initial kernel scaffold; baseline (speedup 1.0000x reference)
import jax
import jax.numpy as jnp
from jax.experimental import pallas as pl


def kernel(sequences, lengths, probs_x, probs_y):
    raise NotImplementedError("write your pallas kernel here")


# speedup vs baseline: 3.2482x; 3.2482x over previous
import jax
import jax.numpy as jnp
from jax import lax
from jax.experimental import pallas as pl
from jax.experimental.pallas import tpu as pltpu

_B, _T, _D, _H = 16, 512, 128, 16


def _fwd_body(seq_ref, len_ref, px_ref, py_ref, out_ref, e_ref, s_ref, m_ref):
    f32 = jnp.float32
    py = jnp.clip(py_ref[...], 1e-5, 1.0 - 1e-5)
    p0 = py[:, 0, :]
    p1 = py[:, 1, :]
    l00 = jnp.log1p(-p0)
    l01 = jnp.log(p0)
    l10 = jnp.log1p(-p1)
    l11 = jnp.log(p1)
    a_t = (l01 - l00).T
    b_t = (l10 - l00).T
    c_t = (l11 - l10 - l01 + l00).T
    base = jnp.sum(l00, axis=1)

    for b in range(_B):
        yb = seq_ref[b]
        ypb = jnp.concatenate([jnp.zeros((1, _D), f32), yb[:-1]], axis=0)
        eb = (jnp.dot(yb, a_t, preferred_element_type=f32)
              + jnp.dot(ypb, b_t, preferred_element_type=f32)
              + jnp.dot(yb * ypb, c_t, preferred_element_type=f32)
              + base[None, :])
        e_ref[:, b, :] = eb

    e_all = e_ref[...]
    m = jnp.max(e_all, axis=2)
    m_ref[...] = m
    e_ref[...] = jnp.exp(e_all - m[:, :, None])

    pmat = jnp.clip(px_ref[...], 1e-6, None)
    p_init = (lax.broadcasted_iota(jnp.int32, (_B, _H), 1) == 0).astype(f32)

    def step(t, p):
        w = e_ref[t]
        pn = jnp.dot(p, pmat, preferred_element_type=f32) * w
        s = jnp.sum(pn, axis=1, keepdims=True)
        s_ref[t] = s[:, 0]
        return pn / s

    lax.fori_loop(0, _T, step, p_init)

    lens = len_ref[...]
    tt = lax.broadcasted_iota(jnp.int32, (_T, _B), 0)
    mask = tt < lens[None, :]
    contrib = jnp.where(mask, m_ref[...] + jnp.log(s_ref[...]), 0.0)
    out_ref[...] = jnp.sum(contrib, axis=0)


def kernel(sequences, lengths, probs_x, probs_y):
    return pl.pallas_call(
        _fwd_body,
        out_shape=jax.ShapeDtypeStruct((_B,), jnp.float32),
        scratch_shapes=[
            pltpu.VMEM((_T, _B, _H), jnp.float32),
            pltpu.VMEM((_T, _B), jnp.float32),
            pltpu.VMEM((_T, _B), jnp.float32),
        ],
    )(sequences, lengths, probs_x.astype(jnp.float32), probs_y)


# speedup vs baseline: 4.9238x; 1.5158x over previous
import jax
import jax.numpy as jnp
from jax import lax
from jax.experimental import pallas as pl
from jax.experimental.pallas import tpu as pltpu

_B, _T, _D, _H = 16, 512, 128, 16


_K = 4
_G = _T // _K


def _fwd_body(seq_ref, len_ref, px_ref, py_ref, out_ref, e_ref, z_ref, m_ref,
              d_ref):
    f32 = jnp.float32
    py = jnp.clip(py_ref[...], 1e-5, 1.0 - 1e-5)
    p0 = py[:, 0, :]
    p1 = py[:, 1, :]
    l00 = jnp.log1p(-p0)
    l01 = jnp.log(p0)
    l10 = jnp.log1p(-p1)
    l11 = jnp.log(p1)
    a_t = (l01 - l00).T.astype(jnp.bfloat16)
    b_t = (l10 - l00).T.astype(jnp.bfloat16)
    c_t = (l11 - l10 - l01 + l00).T.astype(jnp.bfloat16)
    base = jnp.sum(l00, axis=1)

    for b in range(_B):
        yb = seq_ref[b].astype(jnp.bfloat16)
        ypb = jnp.concatenate([jnp.zeros((1, _D), jnp.bfloat16), yb[:-1]], axis=0)
        eb = (jnp.dot(yb, a_t, preferred_element_type=f32)
              + jnp.dot(ypb, b_t, preferred_element_type=f32)
              + jnp.dot(yb * ypb, c_t, preferred_element_type=f32)
              + base[None, :])
        e_ref[:, b, :] = eb

    e_all = e_ref[...]
    m = jnp.max(e_all, axis=2)
    m_ref[...] = m
    e_ref[...] = jnp.exp(e_all - m[:, :, None])

    pmat = jnp.clip(px_ref[...], 1e-6, None)
    p_init = (lax.broadcasted_iota(jnp.int32, (_B, _H), 1) == 0).astype(f32)
    lens = len_ref[...]
    n_groups = (jnp.max(lens) + _K - 1) // _K

    def group(g, p):
        t0 = g * _K
        for i in range(_K):
            w = e_ref[t0 + i]
            p = jnp.dot(p, pmat, preferred_element_type=f32) * w
            z_ref[t0 + i] = jnp.sum(p, axis=1)
        zl = jnp.sum(p, axis=1, keepdims=True)
        d_ref[g] = zl[:, 0]
        return p / zl

    lax.fori_loop(0, n_groups, group, p_init)

    tt = lax.broadcasted_iota(jnp.int32, (_T, _B), 0)
    mask = tt < lens[None, :]
    msum = jnp.sum(jnp.where(mask, m_ref[...], 0.0), axis=0)
    zterm = jnp.sum(jnp.where(tt == lens[None, :] - 1, jnp.log(z_ref[...]), 0.0),
                    axis=0)
    gg = lax.broadcasted_iota(jnp.int32, (_G, _B), 0)
    dmask = (gg + 1) * _K < lens[None, :]
    dterm = jnp.sum(jnp.where(dmask, jnp.log(d_ref[...]), 0.0), axis=0)
    out_ref[...] = msum + zterm + dterm


def kernel(sequences, lengths, probs_x, probs_y):
    return pl.pallas_call(
        _fwd_body,
        out_shape=jax.ShapeDtypeStruct((_B,), jnp.float32),
        scratch_shapes=[
            pltpu.VMEM((_T, _B, _H), jnp.float32),
            pltpu.VMEM((_T, _B), jnp.float32),
            pltpu.VMEM((_T, _B), jnp.float32),
            pltpu.VMEM((_G, _B), jnp.float32),
        ],
    )(sequences, lengths, probs_x.astype(jnp.float32), probs_y)
